# Initial kernel scaffold; baseline (speedup 1.0000x reference)
import jax
import jax.numpy as jnp
from jax.experimental import pallas as pl


def kernel(x, edge_index, W, att_src, att_dst, bias):
    raise NotImplementedError("write your pallas kernel here")


# speedup vs baseline: 22.1376x; 22.1376x over previous
import functools

import jax
import jax.numpy as jnp
from jax import lax
from jax.experimental import pallas as pl
from jax.experimental.pallas import tpu as pltpu
from jax.experimental.pallas import tpu_sc as plsc

N = 10000
E = 320000
C = 128
NPAD = 10240
NW = 32
EW = E // NW
K = 80
CHUNKS = EW // K
ROWS_W = NPAD // NW
ROWS_S = NPAD // 16


def _phase_a(xp, W, att_s, att_d):
    def body(x_ref, w_ref, s_ref, d_ref, xw_ref, asd_ref):
        xw = jnp.dot(x_ref[...], w_ref[...], preferred_element_type=jnp.float32)
        xw_ref[...] = xw
        s = jnp.sum(xw * s_ref[...], axis=1)
        d = jnp.sum(xw * d_ref[...], axis=1)
        asd_ref[...] = jnp.stack([s, d], axis=0)

    return pl.pallas_call(
        body,
        grid=(NPAD // 1024,),
        in_specs=[
            pl.BlockSpec((1024, C), lambda i: (i, 0)),
            pl.BlockSpec((C, C), lambda i: (0, 0)),
            pl.BlockSpec((1, C), lambda i: (0, 0)),
            pl.BlockSpec((1, C), lambda i: (0, 0)),
        ],
        out_specs=[
            pl.BlockSpec((1024, C), lambda i: (i, 0)),
            pl.BlockSpec((2, 1024), lambda i: (0, i)),
        ],
        out_shape=[
            jax.ShapeDtypeStruct((NPAD, C), jnp.float32),
            jax.ShapeDtypeStruct((2, NPAD), jnp.float32),
        ],
    )(xp, W, att_s, att_d)


def _edge_kernel(xw, asd, src_r, dst_r):
    mesh = plsc.VectorSubcoreMesh(core_axis_name="c", subcore_axis_name="s")

    @functools.partial(
        pl.kernel,
        mesh=mesh,
        out_type=[
            jax.ShapeDtypeStruct((2, NPAD, C), jnp.float32),
            jax.ShapeDtypeStruct((2, NPAD), jnp.float32),
        ],
        compiler_params=pltpu.CompilerParams(needs_layout_passes=False),
        scratch_types=[
            pltpu.VMEM((NPAD,), jnp.float32),
            pltpu.VMEM((NPAD,), jnp.float32),
            pltpu.VMEM((K,), jnp.int32),
            pltpu.VMEM((K,), jnp.int32),
            pltpu.VMEM((128,), jnp.float32),
            pltpu.VMEM((K, C), jnp.float32),
            pltpu.VMEM_SHARED((NPAD, C), jnp.float32),
            pltpu.VMEM_SHARED((NPAD,), jnp.float32),
        ],
    )
    def k(xw_hbm, asd_hbm, src_hbm, dst_hbm, accout, denout,
          a_src_v, a_dst_v, src_c, dst_c, p_buf, rows_buf,
          acc_sp, den_sp):
        cid = lax.axis_index("c")
        sid = lax.axis_index("s")
        wid = cid * 16 + sid

        pltpu.sync_copy(asd_hbm.at[0], a_src_v)
        pltpu.sync_copy(asd_hbm.at[1], a_dst_v)

        def zrow(r, carry):
            for c in range(C // 16):
                rows_buf[r, pl.ds(c * 16, 16)] = jnp.zeros((16,), jnp.float32)
            return carry
        lax.fori_loop(0, K, zrow, 0)
        r0 = wid * ROWS_W
        for t in range(ROWS_W // K):
            pltpu.sync_copy(rows_buf, acc_sp.at[pl.ds(r0 + t * K, K)])
        pltpu.sync_copy(rows_buf.at[0], den_sp.at[pl.ds(r0, 128)])
        pltpu.sync_copy(rows_buf.at[0], den_sp.at[pl.ds(r0 + 128, 128)])
        pltpu.sync_copy(rows_buf.at[0, pl.ds(0, 64)], den_sp.at[pl.ds(r0 + 256, 64)])
        plsc.subcore_barrier()

        def amax_body(i, av):
            return jnp.maximum(av, a_src_v[pl.ds(i * 16, 16)])
        avec = lax.fori_loop(0, NPAD // 16, amax_body,
                             jnp.full((16,), -1e30, jnp.float32))
        for s in (1, 2, 4, 8):
            p_buf[pl.ds(0, 16)] = avec
            perm = (jnp.arange(16, dtype=jnp.int32) + s) % 16
            avec = jnp.maximum(avec, plsc.load_gather(p_buf, [perm]))
        amax = avec

        def chunk_body(j, carry):
            pltpu.sync_copy(src_hbm.at[wid, j], src_c)
            pltpu.sync_copy(dst_hbm.at[wid, j], dst_c)
            pltpu.sync_copy(xw_hbm.at[src_c], rows_buf)
            for g in range(K // 16):
                si = src_c[pl.ds(g * 16, 16)]
                di = dst_c[pl.ds(g * 16, 16)]
                s16 = plsc.load_gather(a_src_v, [si])
                d16 = plsc.load_gather(a_dst_v, [di])
                al = s16 + d16
                al = jnp.where(al > 0, al, 0.2 * al)
                mb = amax + d16
                mb = jnp.where(mb > 0, mb, 0.2 * mb)
                p_buf[pl.ds(g * 16, 16)] = jnp.exp(al - mb)
            pltpu.sync_copy(p_buf.at[pl.ds(0, K)], den_sp.at[dst_c], add=True)

            def srow(r, carry2):
                pr = plsc.load_gather(p_buf, [jnp.full((16,), r, jnp.int32)])
                for c in range(C // 16):
                    rows_buf[r, pl.ds(c * 16, 16)] = (
                        rows_buf[r, pl.ds(c * 16, 16)] * pr)
                return carry2
            lax.fori_loop(0, K, srow, 0)
            pltpu.sync_copy(rows_buf, acc_sp.at[dst_c], add=True)
            return carry
        lax.fori_loop(0, CHUNKS, chunk_body, 0)

        plsc.subcore_barrier()
        rb = sid * ROWS_S
        pltpu.sync_copy(acc_sp.at[pl.ds(rb, ROWS_S)],
                        accout.at[cid, pl.ds(rb, ROWS_S)])
        pltpu.sync_copy(den_sp.at[pl.ds(rb, ROWS_S)],
                        denout.at[cid, pl.ds(rb, ROWS_S)])

    return k(xw, asd, src_r, dst_r)


def _phase_c(accout, denout, bias2d):
    def body(acc_ref, den_ref, b_ref, out_ref):
        a = acc_ref[0] + acc_ref[1]
        d = den_ref[0] + den_ref[1] + 1e-16
        out_ref[...] = a / d[:, None] + b_ref[...]

    return pl.pallas_call(
        body,
        grid=(NPAD // 1024,),
        in_specs=[
            pl.BlockSpec((2, 1024, C), lambda i: (0, i, 0)),
            pl.BlockSpec((2, 1024), lambda i: (0, i)),
            pl.BlockSpec((1, C), lambda i: (0, 0)),
        ],
        out_specs=pl.BlockSpec((1024, C), lambda i: (i, 0)),
        out_shape=jax.ShapeDtypeStruct((NPAD, C), jnp.float32),
    )(accout, denout, bias2d)


def kernel(x, edge_index, W, att_src, att_dst, bias):
    xp = jnp.pad(x, ((0, NPAD - N), (0, 0)))
    att_s = att_src.reshape(1, C)
    att_d = att_dst.reshape(1, C)
    xw, asd = _phase_a(xp, W, att_s, att_d)
    src_r = edge_index[0].reshape(NW, CHUNKS, K)
    dst_r = edge_index[1].reshape(NW, CHUNKS, K)
    accout, denout = _edge_kernel(xw, asd, src_r, dst_r)
    out = _phase_c(accout, denout, bias.reshape(1, C))
    return out[:N]
